# Initial kernel scaffold; baseline (speedup 1.0000x reference)
import jax
import jax.numpy as jnp
from jax.experimental import pallas as pl


def kernel(features, image_size, W1, b1, Wc, bc, Wb, bb):
    raise NotImplementedError("write your pallas kernel here")


# speedup vs baseline: 5.3380x; 5.3380x over previous
import numpy as np
import jax
import jax.numpy as jnp
from jax.experimental import pallas as pl

H = 48
W = 48
HW = H * W
C = 256
NA = 9
NTOT = HW * NA
PRE = 500
POST = 50
K5 = 512
IOU_T = 0.7
ONE_BITS = 0x3F800000

_SQ05 = float(np.sqrt(np.float32(0.5)))
_SQ2 = float(np.sqrt(np.float32(2.0)))


def _rpn_kernel(x_ref, w1_ref, b1_ref, wh_ref, bh_ref, out_ref):
    f32 = jnp.float32
    x = x_ref[...]

    y = jnp.zeros((HW, C), f32)
    prow1 = jax.lax.broadcasted_iota(jnp.int32, (HW, 1), 0)
    hh0 = prow1 // W
    ww0 = prow1 - hh0 * W
    for ky in range(3):
        for kx in range(3):
            dy, dx = ky - 1, kx - 1
            o = dy * W + dx
            if o != 0:
                xs = jnp.concatenate([x[o:], x[:o]], axis=0)
            else:
                xs = x
            valid = ((hh0 + dy >= 0) & (hh0 + dy < H) &
                     (ww0 + dx >= 0) & (ww0 + dx < W)).astype(f32)
            y = y + jnp.dot(xs * valid, w1_ref[ky * 3 + kx],
                            preferred_element_type=f32)
    y = jnp.maximum(y + b1_ref[...], 0.0)

    logits = jnp.dot(y, wh_ref[...], preferred_element_type=f32) + bh_ref[...]
    scores = 1.0 / (1.0 + jnp.exp(-logits[:, 0:NA]))

    prow = jax.lax.broadcasted_iota(jnp.int32, (HW, NA), 0)
    acol = jax.lax.broadcasted_iota(jnp.int32, (HW, NA), 1)
    lin = prow * NA + acol
    a2 = lin // HW
    q = lin - a2 * HW
    s_idx = a2 // 3
    r_idx = a2 - s_idx * 3
    scale = jnp.where(s_idx == 0, 8.0, jnp.where(s_idx == 1, 16.0, 32.0))
    sr = jnp.where(r_idx == 0, _SQ05, jnp.where(r_idx == 1, 1.0, _SQ2))
    wa = scale * sr
    ha = scale / sr
    fy = q // W
    fx = q - fy * W
    cx = (fx.astype(f32) + 0.5) * 8.0
    cy = (fy.astype(f32) + 0.5) * 8.0
    ax1 = cx - wa * 0.5
    ay1 = cy - ha * 0.5
    ax2 = cx + wa * 0.5
    ay2 = cy + ha * 0.5

    dxc = jnp.concatenate([logits[:, NA + 4 * a:NA + 4 * a + 1]
                           for a in range(NA)], axis=1)
    dyc = jnp.concatenate([logits[:, NA + 4 * a + 1:NA + 4 * a + 2]
                           for a in range(NA)], axis=1)
    dwc = jnp.concatenate([logits[:, NA + 4 * a + 2:NA + 4 * a + 3]
                           for a in range(NA)], axis=1)
    dhc = jnp.concatenate([logits[:, NA + 4 * a + 3:NA + 4 * a + 4]
                           for a in range(NA)], axis=1)

    widths = jnp.maximum(ax2 - ax1, 1.0)
    heights = jnp.maximum(ay2 - ay1, 1.0)
    ctrx = ax1 + 0.5 * widths
    ctry = ay1 + 0.5 * heights
    pcx = ctrx + dxc * widths
    pcy = ctry + dyc * heights
    pw = widths * jnp.exp(jnp.minimum(dwc, 4.0))
    ph = heights * jnp.exp(jnp.minimum(dhc, 4.0))
    px1 = pcx - 0.5 * pw
    py1 = pcy - 0.5 * ph
    px2 = pcx + 0.5 * pw
    py2 = pcy + 0.5 * ph

    sbits = jax.lax.bitcast_convert_type(scores, jnp.int32)

    def _count_gt(t):
        return jnp.sum((sbits > t).astype(jnp.int32))

    def _bs1(_, lohi):
        lo, hi = lohi
        mid = (lo + hi) // 2
        below = _count_gt(mid) < PRE
        return (jnp.where(below, lo, mid + 1), jnp.where(below, mid, hi))

    tbits, _ = jax.lax.fori_loop(0, 31, _bs1,
                                 (jnp.int32(0), jnp.int32(ONE_BITS)))
    cgt = _count_gt(tbits)
    m = PRE - cgt
    iseq = sbits == tbits

    def _bs2(_, lohi):
        lo, hi = lohi
        mid = (lo + hi) // 2
        geq = jnp.sum((iseq & (lin < mid)).astype(jnp.int32)) >= m
        return (jnp.where(geq, lo, mid + 1), jnp.where(geq, mid, hi))

    ustar, _ = jax.lax.fori_loop(0, 16, _bs2,
                                 (jnp.int32(0), jnp.int32(NTOT)))
    sel = (sbits > tbits) | (iseq & (lin < ustar))
    self32 = sel.astype(f32)

    rowsum = jnp.sum(self32, axis=1, keepdims=True)
    acc = rowsum
    s = 1
    while s < HW:
        acc = acc + jnp.concatenate(
            [jnp.zeros((s, 1), f32), acc[:-s]], axis=0)
        s *= 2
    rowpref = acc - rowsum
    ur = jax.lax.broadcasted_iota(jnp.int32, (NA, NA), 0)
    uc = jax.lax.broadcasted_iota(jnp.int32, (NA, NA), 1)
    u9 = (ur < uc).astype(f32)
    slot = rowpref + jnp.dot(self32, u9, preferred_element_type=f32,
                             precision=jax.lax.Precision.HIGHEST)

    kiota = jax.lax.broadcasted_iota(jnp.int32, (1, K5), 1).astype(f32)
    linf = lin.astype(f32)
    compactT = jnp.zeros((8, K5), f32)
    for a in range(NA):
        va = jnp.concatenate(
            [px1[:, a:a + 1], py1[:, a:a + 1], px2[:, a:a + 1],
             py2[:, a:a + 1], scores[:, a:a + 1], linf[:, a:a + 1],
             jnp.zeros((HW, 2), f32)], axis=1)
        ba = ((slot[:, a:a + 1] == kiota) & sel[:, a:a + 1]).astype(f32)
        compactT = compactT + jax.lax.dot_general(
            va, ba, (((0,), (0,)), ((), ())), preferred_element_type=f32,
            precision=jax.lax.Precision.HIGHEST)

    is_pad = kiota >= float(PRE)
    srow = jnp.where(is_pad, -1.0, compactT[4:5, :])
    irow = jnp.where(is_pad, 40000.0 + kiota, compactT[5:6, :])

    r0 = jax.lax.broadcasted_iota(jnp.int32, (K5, K5), 0)
    c0 = jax.lax.broadcasted_iota(jnp.int32, (K5, K5), 1)
    eye = (r0 == c0).astype(f32)
    scol = jax.lax.dot_general(eye, srow, (((1,), (1,)), ((), ())),
                               preferred_element_type=f32,
                               precision=jax.lax.Precision.HIGHEST)
    icol = jax.lax.dot_general(eye, irow, (((1,), (1,)), ((), ())),
                               preferred_element_type=f32,
                               precision=jax.lax.Precision.HIGHEST)
    beats = ((scol > srow) | ((scol == srow) & (icol < irow))).astype(f32)
    rank = jnp.sum(beats, axis=0, keepdims=True)
    rmat = (jax.lax.broadcasted_iota(jnp.int32, (K5, K5), 0).astype(f32) == rank).astype(f32)
    sortedT = jax.lax.dot_general(
        compactT, rmat, (((1,), (1,)), ((), ())), preferred_element_type=f32,
        precision=jax.lax.Precision.HIGHEST)

    xr = sortedT[0:1, :]
    yr = sortedT[1:2, :]
    x2r = sortedT[2:3, :]
    y2r = sortedT[3:4, :]
    area = (x2r - xr) * (y2r - yr)
    lane = jax.lax.broadcasted_iota(jnp.int32, (1, K5), 1).astype(f32)
    keep0 = (lane < float(PRE)).astype(f32)

    def _nms(i, keep):
        fi = i.astype(f32)
        onei = (lane == fi).astype(f32)
        xi = jnp.sum(xr * onei)
        yi = jnp.sum(yr * onei)
        x2i = jnp.sum(x2r * onei)
        y2i = jnp.sum(y2r * onei)
        ki = jnp.sum(keep * onei)
        ai = (x2i - xi) * (y2i - yi)
        inter = (jnp.maximum(jnp.minimum(x2r, x2i) - jnp.maximum(xr, xi), 0.0)
                 * jnp.maximum(jnp.minimum(y2r, y2i) - jnp.maximum(yr, yi),
                               0.0))
        iou = inter / jnp.maximum(area + ai - inter, 1e-6)
        sup = ((ki > 0.0) & (iou > IOU_T) & (lane > fi)).astype(f32)
        return keep * (1.0 - sup)

    keep = jax.lax.fori_loop(0, PRE, _nms, keep0)

    umat = (r0 < c0).astype(f32)
    pos = jnp.dot(keep, umat, preferred_element_type=f32,
                  precision=jax.lax.Precision.HIGHEST)
    jio = jax.lax.broadcasted_iota(jnp.int32, (128, K5), 0).astype(f32)
    amat = ((jio == pos) & (keep > 0.0) & (pos < float(POST))).astype(f32)
    outT = jax.lax.dot_general(
        sortedT, amat, (((1,), (1,)), ((), ())), preferred_element_type=f32,
        precision=jax.lax.Precision.HIGHEST)
    nkept = jnp.sum(keep)
    jr = jax.lax.broadcasted_iota(jnp.int32, (1, 128), 1).astype(f32)
    fill = (jr >= nkept).astype(f32)
    out_ref[...] = outT + sortedT[:, 0:1] * fill


def kernel(features, image_size, W1, b1, Wc, bc, Wb, bb):
    del image_size
    x = jnp.transpose(features[0], (1, 2, 0)).reshape(HW, C)
    w1r = jnp.transpose(W1, (2, 3, 1, 0)).reshape(9, C, C)
    wc2 = jnp.transpose(Wc[:, :, 0, 0], (1, 0))
    wb2 = jnp.transpose(Wb[:, :, 0, 0], (1, 0))
    whead = jnp.concatenate(
        [wc2, wb2, jnp.zeros((C, 128 - 45), jnp.float32)], axis=1)
    bhead = jnp.concatenate(
        [bc, bb, jnp.zeros((128 - 45,), jnp.float32)])[None, :]
    outT = pl.pallas_call(
        _rpn_kernel,
        out_shape=jax.ShapeDtypeStruct((8, 128), jnp.float32),
    )(x, w1r, b1[None, :], whead, bhead)
    return outT.T[:POST, :4]


# speedup vs baseline: 5.6978x; 1.0674x over previous
import numpy as np
import jax
import jax.numpy as jnp
from jax.experimental import pallas as pl
from jax.experimental.pallas import tpu as pltpu

H = 48
W = 48
HW = H * W
C = 256
NA = 9
NTOT = HW * NA
PRE = 500
POST = 50
K5 = 512
IOU_T = 0.7
ONE_BITS = 0x3F800000

_SQ05 = float(np.sqrt(np.float32(0.5)))
_SQ2 = float(np.sqrt(np.float32(2.0)))


def _rpn_kernel(x_ref, w1_ref, b1_ref, wh_ref, bh_ref, out_ref, sup_ref):
    f32 = jnp.float32
    x = x_ref[...]

    y = jnp.zeros((HW, C), f32)
    prow1 = jax.lax.broadcasted_iota(jnp.int32, (HW, 1), 0)
    hh0 = prow1 // W
    ww0 = prow1 - hh0 * W
    for ky in range(3):
        for kx in range(3):
            dy, dx = ky - 1, kx - 1
            o = dy * W + dx
            if o != 0:
                xs = jnp.concatenate([x[o:], x[:o]], axis=0)
            else:
                xs = x
            valid = ((hh0 + dy >= 0) & (hh0 + dy < H) &
                     (ww0 + dx >= 0) & (ww0 + dx < W)).astype(f32)
            y = y + jnp.dot(xs * valid, w1_ref[ky * 3 + kx],
                            preferred_element_type=f32)
    y = jnp.maximum(y + b1_ref[...], 0.0)

    logits = jnp.dot(y, wh_ref[...], preferred_element_type=f32) + bh_ref[...]
    scores = 1.0 / (1.0 + jnp.exp(-logits[:, 0:NA]))

    prow = jax.lax.broadcasted_iota(jnp.int32, (HW, NA), 0)
    acol = jax.lax.broadcasted_iota(jnp.int32, (HW, NA), 1)
    lin = prow * NA + acol

    dxc = jnp.concatenate([logits[:, NA + 4 * a:NA + 4 * a + 1]
                           for a in range(NA)], axis=1)
    dyc = jnp.concatenate([logits[:, NA + 4 * a + 1:NA + 4 * a + 2]
                           for a in range(NA)], axis=1)
    dwc = jnp.concatenate([logits[:, NA + 4 * a + 2:NA + 4 * a + 3]
                           for a in range(NA)], axis=1)
    dhc = jnp.concatenate([logits[:, NA + 4 * a + 3:NA + 4 * a + 4]
                           for a in range(NA)], axis=1)

    sbits = jax.lax.bitcast_convert_type(scores, jnp.int32)

    def _count_gt(t):
        return jnp.sum((sbits > t).astype(jnp.int32))

    def _bs1(_, lohi):
        lo, hi = lohi
        mid = (lo + hi) // 2
        below = _count_gt(mid) < PRE
        return (jnp.where(below, lo, mid + 1), jnp.where(below, mid, hi))

    tbits, _ = jax.lax.fori_loop(0, 31, _bs1,
                                 (jnp.int32(0), jnp.int32(ONE_BITS)))
    cgt = _count_gt(tbits)
    m = PRE - cgt
    iseq = sbits == tbits

    def _bs2(_, lohi):
        lo, hi = lohi
        mid = (lo + hi) // 2
        geq = jnp.sum((iseq & (lin < mid)).astype(jnp.int32)) >= m
        return (jnp.where(geq, lo, mid + 1), jnp.where(geq, mid, hi))

    ustar, _ = jax.lax.fori_loop(0, 16, _bs2,
                                 (jnp.int32(0), jnp.int32(NTOT)))
    sel = (sbits > tbits) | (iseq & (lin < ustar))
    self32 = sel.astype(f32)

    rowsum = jnp.sum(self32, axis=1, keepdims=True)
    acc = rowsum
    s = 1
    while s < HW:
        acc = acc + jnp.concatenate(
            [jnp.zeros((s, 1), f32), acc[:-s]], axis=0)
        s *= 2
    rowpref = acc - rowsum
    ur = jax.lax.broadcasted_iota(jnp.int32, (NA, NA), 0)
    uc = jax.lax.broadcasted_iota(jnp.int32, (NA, NA), 1)
    u9 = (ur < uc).astype(f32)
    slot = rowpref + jnp.dot(self32, u9, preferred_element_type=f32,
                             precision=jax.lax.Precision.HIGHEST)

    kiota = jax.lax.broadcasted_iota(jnp.int32, (1, K5), 1).astype(f32)
    linf = lin.astype(f32)
    compactT = jnp.zeros((8, K5), f32)
    for a in range(NA):
        va = jnp.concatenate(
            [dxc[:, a:a + 1], dyc[:, a:a + 1], dwc[:, a:a + 1],
             dhc[:, a:a + 1], scores[:, a:a + 1], linf[:, a:a + 1],
             jnp.zeros((HW, 2), f32)], axis=1)
        ba = ((slot[:, a:a + 1] == kiota) & sel[:, a:a + 1]).astype(f32)
        compactT = compactT + jax.lax.dot_general(
            va, ba, (((0,), (0,)), ((), ())), preferred_element_type=f32,
            precision=jax.lax.Precision.HIGHEST)

    is_pad = kiota >= float(PRE)
    srow = jnp.where(is_pad, -1.0, compactT[4:5, :])
    irow = jnp.where(is_pad, 40000.0 + kiota, compactT[5:6, :])

    r0 = jax.lax.broadcasted_iota(jnp.int32, (K5, K5), 0)
    c0 = jax.lax.broadcasted_iota(jnp.int32, (K5, K5), 1)
    eye = (r0 == c0).astype(f32)
    scol = jax.lax.dot_general(eye, srow, (((1,), (1,)), ((), ())),
                               preferred_element_type=f32,
                               precision=jax.lax.Precision.HIGHEST)
    icol = jax.lax.dot_general(eye, irow, (((1,), (1,)), ((), ())),
                               preferred_element_type=f32,
                               precision=jax.lax.Precision.HIGHEST)
    beats = ((scol > srow) | ((scol == srow) & (icol < irow))).astype(f32)
    rank = jnp.sum(beats, axis=0, keepdims=True)
    rmat = (jax.lax.broadcasted_iota(jnp.int32, (K5, K5), 0).astype(f32) == rank).astype(f32)
    sortedT = jax.lax.dot_general(
        compactT, rmat, (((1,), (1,)), ((), ())), preferred_element_type=f32,
        precision=jax.lax.Precision.HIGHEST)

    idxr = sortedT[5:6, :].astype(jnp.int32)
    a2 = idxr // HW
    q = idxr - a2 * HW
    s_idx = a2 // 3
    r_idx = a2 - s_idx * 3
    scale = jnp.where(s_idx == 0, 8.0, jnp.where(s_idx == 1, 16.0, 32.0))
    sr = jnp.where(r_idx == 0, _SQ05, jnp.where(r_idx == 1, 1.0, _SQ2))
    wa = scale * sr
    ha = scale / sr
    fy = q // W
    fx = q - fy * W
    cx = (fx.astype(f32) + 0.5) * 8.0
    cy = (fy.astype(f32) + 0.5) * 8.0
    ax1 = cx - wa * 0.5
    ay1 = cy - ha * 0.5
    ax2 = cx + wa * 0.5
    ay2 = cy + ha * 0.5
    widths = jnp.maximum(ax2 - ax1, 1.0)
    heights = jnp.maximum(ay2 - ay1, 1.0)
    ctrx = ax1 + 0.5 * widths
    ctry = ay1 + 0.5 * heights
    pcx = ctrx + sortedT[0:1, :] * widths
    pcy = ctry + sortedT[1:2, :] * heights
    pw = widths * jnp.exp(jnp.minimum(sortedT[2:3, :], 4.0))
    ph = heights * jnp.exp(jnp.minimum(sortedT[3:4, :], 4.0))
    xr = pcx - 0.5 * pw
    yr = pcy - 0.5 * ph
    x2r = pcx + 0.5 * pw
    y2r = pcy + 0.5 * ph

    xc = jax.lax.dot_general(eye, xr, (((1,), (1,)), ((), ())),
                             preferred_element_type=f32,
                             precision=jax.lax.Precision.HIGHEST)
    yc = jax.lax.dot_general(eye, yr, (((1,), (1,)), ((), ())),
                             preferred_element_type=f32,
                             precision=jax.lax.Precision.HIGHEST)
    x2c = jax.lax.dot_general(eye, x2r, (((1,), (1,)), ((), ())),
                              preferred_element_type=f32,
                              precision=jax.lax.Precision.HIGHEST)
    y2c = jax.lax.dot_general(eye, y2r, (((1,), (1,)), ((), ())),
                              preferred_element_type=f32,
                              precision=jax.lax.Precision.HIGHEST)
    area_r = (x2r - xr) * (y2r - yr)
    area_c = (x2c - xc) * (y2c - yc)
    inter = (jnp.maximum(jnp.minimum(x2c, x2r) - jnp.maximum(xc, xr), 0.0)
             * jnp.maximum(jnp.minimum(y2c, y2r) - jnp.maximum(yc, yr), 0.0))
    iou = inter / jnp.maximum(area_c + area_r - inter, 1e-6)
    sup_ref[...] = ((iou > IOU_T) & (r0 < c0)).astype(f32)
    lane = jax.lax.broadcasted_iota(jnp.int32, (1, K5), 1).astype(f32)
    keep0 = (lane < float(PRE)).astype(f32)

    def _nms(i, keep):
        ki = jnp.sum(keep * (lane == i.astype(f32)).astype(f32))
        row = sup_ref[pl.ds(i, 1), :]
        return keep * (1.0 - ki * row)

    keep = jax.lax.fori_loop(0, PRE, _nms, keep0)

    umat = (r0 < c0).astype(f32)
    pos = jnp.dot(keep, umat, preferred_element_type=f32,
                  precision=jax.lax.Precision.HIGHEST)
    jio = jax.lax.broadcasted_iota(jnp.int32, (128, K5), 0).astype(f32)
    amat = ((jio == pos) & (keep > 0.0) & (pos < float(POST))).astype(f32)
    boxT = jnp.concatenate([xr, yr, x2r, y2r, jnp.zeros((4, K5), f32)],
                           axis=0)
    outT = jax.lax.dot_general(
        boxT, amat, (((1,), (1,)), ((), ())), preferred_element_type=f32,
        precision=jax.lax.Precision.HIGHEST)
    nkept = jnp.sum(keep)
    jr = jax.lax.broadcasted_iota(jnp.int32, (1, 128), 1).astype(f32)
    fill = (jr >= nkept).astype(f32)
    out_ref[...] = outT + boxT[:, 0:1] * fill


def kernel(features, image_size, W1, b1, Wc, bc, Wb, bb):
    del image_size
    x = jnp.transpose(features[0], (1, 2, 0)).reshape(HW, C)
    w1r = jnp.transpose(W1, (2, 3, 1, 0)).reshape(9, C, C)
    wc2 = jnp.transpose(Wc[:, :, 0, 0], (1, 0))
    wb2 = jnp.transpose(Wb[:, :, 0, 0], (1, 0))
    whead = jnp.concatenate(
        [wc2, wb2, jnp.zeros((C, 128 - 45), jnp.float32)], axis=1)
    bhead = jnp.concatenate(
        [bc, bb, jnp.zeros((128 - 45,), jnp.float32)])[None, :]
    outT = pl.pallas_call(
        _rpn_kernel,
        out_shape=jax.ShapeDtypeStruct((8, 128), jnp.float32),
        scratch_shapes=[pltpu.VMEM((K5, K5), jnp.float32)],
    )(x, w1r, b1[None, :], whead, bhead)
    return outT.T[:POST, :4]


# speedup vs baseline: 10.0306x; 1.7604x over previous
import numpy as np
import jax
import jax.numpy as jnp
from jax.experimental import pallas as pl
from jax.experimental.pallas import tpu as pltpu

H = 48
W = 48
HW = H * W
C = 256
NA = 9
NTOT = HW * NA
PRE = 500
POST = 50
K5 = 512
IOU_T = 0.7
ONE_BITS = 0x3F800000

_SQ05 = float(np.sqrt(np.float32(0.5)))
_SQ2 = float(np.sqrt(np.float32(2.0)))


def _rpn_kernel(x_ref, w1_ref, b1_ref, wh_ref, bh_ref, out_ref):
    f32 = jnp.float32
    x = x_ref[...]

    y = jnp.zeros((HW, C), f32)
    prow1 = jax.lax.broadcasted_iota(jnp.int32, (HW, 1), 0)
    hh0 = prow1 // W
    ww0 = prow1 - hh0 * W
    for ky in range(3):
        for kx in range(3):
            dy, dx = ky - 1, kx - 1
            o = dy * W + dx
            if o != 0:
                xs = jnp.concatenate([x[o:], x[:o]], axis=0)
            else:
                xs = x
            valid = ((hh0 + dy >= 0) & (hh0 + dy < H) &
                     (ww0 + dx >= 0) & (ww0 + dx < W)).astype(f32)
            y = y + jnp.dot(xs * valid, w1_ref[ky * 3 + kx],
                            preferred_element_type=f32)
    y = jnp.maximum(y + b1_ref[...], 0.0)

    logits = jnp.dot(y, wh_ref[...], preferred_element_type=f32) + bh_ref[...]
    scores = 1.0 / (1.0 + jnp.exp(-logits[:, 0:NA]))

    prow = jax.lax.broadcasted_iota(jnp.int32, (HW, NA), 0)
    acol = jax.lax.broadcasted_iota(jnp.int32, (HW, NA), 1)
    lin = prow * NA + acol

    dxc = jnp.concatenate([logits[:, NA + 4 * a:NA + 4 * a + 1]
                           for a in range(NA)], axis=1)
    dyc = jnp.concatenate([logits[:, NA + 4 * a + 1:NA + 4 * a + 2]
                           for a in range(NA)], axis=1)
    dwc = jnp.concatenate([logits[:, NA + 4 * a + 2:NA + 4 * a + 3]
                           for a in range(NA)], axis=1)
    dhc = jnp.concatenate([logits[:, NA + 4 * a + 3:NA + 4 * a + 4]
                           for a in range(NA)], axis=1)

    sbits = jax.lax.bitcast_convert_type(scores, jnp.int32)

    def _count_gt(t):
        return jnp.sum((sbits > t).astype(jnp.int32))

    def _bs1(_, lohi):
        lo, hi = lohi
        mid = (lo + hi) // 2
        below = _count_gt(mid) < PRE
        return (jnp.where(below, lo, mid + 1), jnp.where(below, mid, hi))

    tbits, _ = jax.lax.fori_loop(0, 31, _bs1,
                                 (jnp.int32(0), jnp.int32(ONE_BITS)))
    cgt = _count_gt(tbits)
    m = PRE - cgt
    iseq = sbits == tbits

    def _bs2(_, lohi):
        lo, hi = lohi
        mid = (lo + hi) // 2
        geq = jnp.sum((iseq & (lin < mid)).astype(jnp.int32)) >= m
        return (jnp.where(geq, lo, mid + 1), jnp.where(geq, mid, hi))

    ustar, _ = jax.lax.fori_loop(0, 16, _bs2,
                                 (jnp.int32(0), jnp.int32(NTOT)))
    sel = (sbits > tbits) | (iseq & (lin < ustar))
    self32 = sel.astype(f32)

    rowsum = jnp.sum(self32, axis=1, keepdims=True)
    acc = rowsum
    s = 1
    while s < HW:
        acc = acc + jnp.concatenate(
            [jnp.zeros((s, 1), f32), acc[:-s]], axis=0)
        s *= 2
    rowpref = acc - rowsum
    ur = jax.lax.broadcasted_iota(jnp.int32, (NA, NA), 0)
    uc = jax.lax.broadcasted_iota(jnp.int32, (NA, NA), 1)
    u9 = (ur < uc).astype(f32)
    slot = rowpref + jnp.dot(self32, u9, preferred_element_type=f32,
                             precision=jax.lax.Precision.HIGHEST)

    kiota = jax.lax.broadcasted_iota(jnp.int32, (1, K5), 1).astype(f32)
    linf = lin.astype(f32)
    compactT = jnp.zeros((8, K5), f32)
    for a in range(NA):
        va = jnp.concatenate(
            [dxc[:, a:a + 1], dyc[:, a:a + 1], dwc[:, a:a + 1],
             dhc[:, a:a + 1], scores[:, a:a + 1], linf[:, a:a + 1],
             jnp.zeros((HW, 2), f32)], axis=1)
        ba = ((slot[:, a:a + 1] == kiota) & sel[:, a:a + 1]).astype(f32)
        compactT = compactT + jax.lax.dot_general(
            va, ba, (((0,), (0,)), ((), ())), preferred_element_type=f32,
            precision=jax.lax.Precision.HIGHEST)

    is_pad = kiota >= float(PRE)
    srow = jnp.where(is_pad, -1.0, compactT[4:5, :])
    irow = jnp.where(is_pad, 40000.0 + kiota, compactT[5:6, :])

    r0 = jax.lax.broadcasted_iota(jnp.int32, (K5, K5), 0)
    c0 = jax.lax.broadcasted_iota(jnp.int32, (K5, K5), 1)
    eye = (r0 == c0).astype(f32)
    scol = jax.lax.dot_general(eye, srow, (((1,), (1,)), ((), ())),
                               preferred_element_type=f32,
                               precision=jax.lax.Precision.HIGHEST)
    icol = jax.lax.dot_general(eye, irow, (((1,), (1,)), ((), ())),
                               preferred_element_type=f32,
                               precision=jax.lax.Precision.HIGHEST)
    beats = ((scol > srow) | ((scol == srow) & (icol < irow))).astype(f32)
    rank = jnp.sum(beats, axis=0, keepdims=True)
    rmat = (jax.lax.broadcasted_iota(jnp.int32, (K5, K5), 0).astype(f32) == rank).astype(f32)
    sortedT = jax.lax.dot_general(
        compactT, rmat, (((1,), (1,)), ((), ())), preferred_element_type=f32,
        precision=jax.lax.Precision.HIGHEST)

    idxr = sortedT[5:6, :].astype(jnp.int32)
    a2 = idxr // HW
    q = idxr - a2 * HW
    s_idx = a2 // 3
    r_idx = a2 - s_idx * 3
    scale = jnp.where(s_idx == 0, 8.0, jnp.where(s_idx == 1, 16.0, 32.0))
    sr = jnp.where(r_idx == 0, _SQ05, jnp.where(r_idx == 1, 1.0, _SQ2))
    wa = scale * sr
    ha = scale / sr
    fy = q // W
    fx = q - fy * W
    cx = (fx.astype(f32) + 0.5) * 8.0
    cy = (fy.astype(f32) + 0.5) * 8.0
    ax1 = cx - wa * 0.5
    ay1 = cy - ha * 0.5
    ax2 = cx + wa * 0.5
    ay2 = cy + ha * 0.5
    widths = jnp.maximum(ax2 - ax1, 1.0)
    heights = jnp.maximum(ay2 - ay1, 1.0)
    ctrx = ax1 + 0.5 * widths
    ctry = ay1 + 0.5 * heights
    pcx = ctrx + sortedT[0:1, :] * widths
    pcy = ctry + sortedT[1:2, :] * heights
    pw = widths * jnp.exp(jnp.minimum(sortedT[2:3, :], 4.0))
    ph = heights * jnp.exp(jnp.minimum(sortedT[3:4, :], 4.0))
    xr = pcx - 0.5 * pw
    yr = pcy - 0.5 * ph
    x2r = pcx + 0.5 * pw
    y2r = pcy + 0.5 * ph

    xc = jax.lax.dot_general(eye, xr, (((1,), (1,)), ((), ())),
                             preferred_element_type=f32,
                             precision=jax.lax.Precision.HIGHEST)
    yc = jax.lax.dot_general(eye, yr, (((1,), (1,)), ((), ())),
                             preferred_element_type=f32,
                             precision=jax.lax.Precision.HIGHEST)
    x2c = jax.lax.dot_general(eye, x2r, (((1,), (1,)), ((), ())),
                              preferred_element_type=f32,
                              precision=jax.lax.Precision.HIGHEST)
    y2c = jax.lax.dot_general(eye, y2r, (((1,), (1,)), ((), ())),
                              preferred_element_type=f32,
                              precision=jax.lax.Precision.HIGHEST)
    area_r = (x2r - xr) * (y2r - yr)
    area_c = (x2c - xc) * (y2c - yc)
    inter = (jnp.maximum(jnp.minimum(x2c, x2r) - jnp.maximum(xc, xr), 0.0)
             * jnp.maximum(jnp.minimum(y2c, y2r) - jnp.maximum(yc, yr), 0.0))
    iou = inter / jnp.maximum(area_c + area_r - inter, 1e-6)
    sup = ((iou > IOU_T) & (r0 < c0)).astype(f32)
    lane = jax.lax.broadcasted_iota(jnp.int32, (1, K5), 1).astype(f32)
    keep0 = (lane < float(PRE)).astype(f32)

    def _cond(carry):
        return carry[1]

    def _sweep(carry):
        k, _ = carry
        t = jnp.dot(k, sup, preferred_element_type=f32)
        k2 = keep0 * (t == 0.0).astype(f32)
        changed = jnp.sum(jnp.abs(k2 - k)) > 0.0
        return (k2, changed)

    keep, _ = jax.lax.while_loop(_cond, _sweep, (keep0, True))

    umat = (r0 < c0).astype(f32)
    pos = jnp.dot(keep, umat, preferred_element_type=f32,
                  precision=jax.lax.Precision.HIGHEST)
    jio = jax.lax.broadcasted_iota(jnp.int32, (128, K5), 0).astype(f32)
    amat = ((jio == pos) & (keep > 0.0) & (pos < float(POST))).astype(f32)
    boxT = jnp.concatenate([xr, yr, x2r, y2r, jnp.zeros((4, K5), f32)],
                           axis=0)
    outT = jax.lax.dot_general(
        boxT, amat, (((1,), (1,)), ((), ())), preferred_element_type=f32,
        precision=jax.lax.Precision.HIGHEST)
    nkept = jnp.sum(keep)
    jr = jax.lax.broadcasted_iota(jnp.int32, (1, 128), 1).astype(f32)
    fill = (jr >= nkept).astype(f32)
    out_ref[...] = outT + boxT[:, 0:1] * fill


def kernel(features, image_size, W1, b1, Wc, bc, Wb, bb):
    del image_size
    x = jnp.transpose(features[0], (1, 2, 0)).reshape(HW, C)
    w1r = jnp.transpose(W1, (2, 3, 1, 0)).reshape(9, C, C)
    wc2 = jnp.transpose(Wc[:, :, 0, 0], (1, 0))
    wb2 = jnp.transpose(Wb[:, :, 0, 0], (1, 0))
    whead = jnp.concatenate(
        [wc2, wb2, jnp.zeros((C, 128 - 45), jnp.float32)], axis=1)
    bhead = jnp.concatenate(
        [bc, bb, jnp.zeros((128 - 45,), jnp.float32)])[None, :]
    outT = pl.pallas_call(
        _rpn_kernel,
        out_shape=jax.ShapeDtypeStruct((8, 128), jnp.float32),
    )(x, w1r, b1[None, :], whead, bhead)
    return outT.T[:POST, :4]
